# baseline (speedup 1.0000x reference)
import functools

import jax
import jax.numpy as jnp
from jax import lax
from jax.experimental import pallas as pl
from jax.experimental.pallas import tpu as pltpu
from jax.experimental.pallas import tpu_sc as plsc

N_NODES = 10000
D = 128
N_PAD = 10240
N_EDGES = 320000
NC = 2
NS = 16
CHUNK = 128
AGG_CHUNK = 64
E_PAD = 327680
NCHUNK = E_PAD // (NC * NS * AGG_CHUNK)
NCHUNK_DEG = E_PAD // (NS * CHUNK)
SLAB = N_PAD // NS
TC_BLK = 640
TC_GRID = N_PAD // TC_BLK


def _deg_body(comb_hbm, ones_hbm, z_hbm, out_hbm,
              comb_v, idx_a, idx_b, ones_v, acc, sem_a, sem_b):
    cid = lax.axis_index("c")
    sid = lax.axis_index("s")
    pltpu.sync_copy(comb_hbm.at[sid], comb_v)
    pltpu.sync_copy(ones_hbm, ones_v)
    row0 = sid * SLAB
    pltpu.sync_copy(z_hbm, acc.at[pl.ds(row0, SLAB)])
    plsc.subcore_barrier()

    def extract(j, dst):
        for k in range(CHUNK // 16):
            v = comb_v[j, pl.ds(k * 16, 16)]
            lo = lax.bitwise_and(v, 0xFFFF)
            hi = lax.shift_right_logical(v, 16)
            dst[0, pl.ds(k * 16, 16)] = jnp.where(cid == 0, lo, hi)

    def scat(idx, sem):
        pltpu.async_copy(ones_v, acc.at[idx.at[0]], sem, add=True)

    def swait(idx, sem):
        pltpu.make_async_copy(ones_v, acc.at[idx.at[0]], sem).wait()

    extract(0, idx_a)
    scat(idx_a, sem_a)

    def body(jj, carry):
        j0 = 2 * jj
        extract(j0 + 1, idx_b)
        scat(idx_b, sem_b)
        swait(idx_a, sem_a)

        @pl.when(j0 + 2 < NCHUNK_DEG)
        def _():
            extract(j0 + 2, idx_a)
            scat(idx_a, sem_a)

        swait(idx_b, sem_b)
        return carry

    lax.fori_loop(0, NCHUNK_DEG // 2, body, 0)
    plsc.subcore_barrier()
    pltpu.sync_copy(acc.at[pl.ds(row0, SLAB)],
                    out_hbm.at[cid, pl.ds(row0, SLAB)])


def _agg_body(comb_hbm, h_hbm, z_hbm, out_hbm,
              comb_v, idx_a, idx_b, rows0, rows1, acc_sh, sem0, sem1):
    cid = lax.axis_index("c")
    sid = lax.axis_index("s")
    pltpu.sync_copy(comb_hbm.at[cid, sid], comb_v)
    row0 = sid * SLAB
    pltpu.sync_copy(z_hbm, acc_sh.at[pl.ds(row0, SLAB)])
    plsc.subcore_barrier()

    def extract(j, dst):
        for k in range(AGG_CHUNK // 16):
            v = comb_v[j, pl.ds(k * 16, 16)]
            dst[0, pl.ds(k * 16, 16)] = lax.bitwise_and(v, 0xFFFF)
            dst[1, pl.ds(k * 16, 16)] = lax.shift_right_logical(v, 16)

    def gather(idx, buf, sem):
        pltpu.async_copy(h_hbm.at[idx.at[0]], buf, sem)

    def gwait(idx, buf, sem):
        pltpu.make_async_copy(h_hbm.at[idx.at[0]], buf, sem).wait()

    extract(0, idx_a)
    gather(idx_a, rows0, sem0)

    def body(jj, carry):
        j0 = 2 * jj
        extract(j0 + 1, idx_b)
        gather(idx_b, rows1, sem1)
        gwait(idx_a, rows0, sem0)
        pltpu.sync_copy(rows0, acc_sh.at[idx_a.at[1]], add=True)

        @pl.when(j0 + 2 < NCHUNK)
        def _():
            extract(j0 + 2, idx_a)
            gather(idx_a, rows0, sem0)

        gwait(idx_b, rows1, sem1)
        pltpu.sync_copy(rows1, acc_sh.at[idx_b.at[1]], add=True)
        return carry

    lax.fori_loop(0, NCHUNK // 2, body, 0)
    plsc.subcore_barrier()
    pltpu.sync_copy(acc_sh.at[pl.ds(row0, SLAB)],
                    out_hbm.at[cid, pl.ds(row0, SLAB)])


def _sc_mesh():
    return plsc.VectorSubcoreMesh(core_axis_name="c", subcore_axis_name="s")


def _deg_call(comb_deg, ones128, z128):
    return pl.kernel(
        _deg_body,
        out_type=jax.ShapeDtypeStruct((NC, N_PAD, D), jnp.float32),
        mesh=_sc_mesh(),
        scratch_types=[
            pltpu.VMEM((NCHUNK_DEG, CHUNK), jnp.int32),
            pltpu.VMEM((1, CHUNK), jnp.int32),
            pltpu.VMEM((1, CHUNK), jnp.int32),
            pltpu.VMEM((CHUNK, D), jnp.float32),
            pltpu.VMEM_SHARED((N_PAD, D), jnp.float32),
            pltpu.SemaphoreType.DMA,
            pltpu.SemaphoreType.DMA,
        ],
    )(comb_deg, ones128, z128)


def _agg_call(comb, h, z128):
    return pl.kernel(
        _agg_body,
        out_type=jax.ShapeDtypeStruct((NC, N_PAD, D), jnp.float32),
        mesh=_sc_mesh(),
        scratch_types=[
            pltpu.VMEM((NCHUNK, AGG_CHUNK), jnp.int32),
            pltpu.VMEM((2, AGG_CHUNK), jnp.int32),
            pltpu.VMEM((2, AGG_CHUNK), jnp.int32),
            pltpu.VMEM((AGG_CHUNK, D), jnp.float32),
            pltpu.VMEM((AGG_CHUNK, D), jnp.float32),
            pltpu.VMEM_SHARED((N_PAD, D), jnp.float32),
            pltpu.SemaphoreType.DMA,
            pltpu.SemaphoreType.DMA,
        ],
    )(comb, h, z128)


def _inv_sqrt_deg(dp, kind):
    deg = dp[kind, :, 0:1].astype(jnp.float32)
    return lax.rsqrt(jnp.maximum(deg, 1.0))


def _mlp(h, w0, b0, w1, b1):
    h = jnp.maximum(jnp.dot(h, w0, preferred_element_type=jnp.float32) + b0, 0.0)
    h = jnp.maximum(jnp.dot(h, w1, preferred_element_type=jnp.float32) + b1, 0.0)
    return h


def _post(p, dp, nodes, lns, lnb):
    agg = (p[0] + p[1]) * _inv_sqrt_deg(dp, 1)
    nd = agg + nodes
    mean = jnp.mean(nd, axis=-1, keepdims=True)
    c = nd - mean
    var = jnp.mean(c * c, axis=-1, keepdims=True)
    return c * lax.rsqrt(var + 1e-6) * lns + lnb


def _ab0_body(x_ref, dp_ref, we, be, w0, b0, w1, b1, nodes_ref, h_ref):
    nodes = jnp.dot(x_ref[...], we[...],
                    preferred_element_type=jnp.float32) + be[...]
    h = _mlp(nodes, w0[...], b0[...], w1[...], b1[...])
    nodes_ref[...] = nodes
    h_ref[...] = h * _inv_sqrt_deg(dp_ref[...], 0)


def _cb_body(p_ref, dp_ref, nodes_ref, lns, lnb, w0, b0, w1, b1,
             nodes_out, h_out):
    dp = dp_ref[...]
    nodes_new = _post(p_ref[...], dp, nodes_ref[...], lns[...], lnb[...])
    h = _mlp(nodes_new, w0[...], b0[...], w1[...], b1[...])
    nodes_out[...] = nodes_new
    h_out[...] = h * _inv_sqrt_deg(dp, 0)


def _cd_body(p_ref, dp_ref, nodes_ref, lns, lnb, wd, bd, out_ref):
    nodes_new = _post(p_ref[...], dp_ref[...], nodes_ref[...], lns[...], lnb[...])
    out_ref[...] = jnp.dot(nodes_new, wd[...],
                           preferred_element_type=jnp.float32) + bd[...]


_spec_row = pl.BlockSpec((TC_BLK, D), lambda i: (i, 0))
_spec_w = pl.BlockSpec((D, D), lambda i: (0, 0))
_spec_b = pl.BlockSpec((1, D), lambda i: (0, 0))
_spec_dp = pl.BlockSpec((NC, TC_BLK, D), lambda i: (0, i, 0))
_spec_p = pl.BlockSpec((NC, TC_BLK, D), lambda i: (0, i, 0))

_row_out = jax.ShapeDtypeStruct((N_PAD, D), jnp.float32)

_ab0 = pl.pallas_call(
    _ab0_body, grid=(TC_GRID,),
    in_specs=[_spec_row, _spec_dp, _spec_w, _spec_b, _spec_w, _spec_b,
              _spec_w, _spec_b],
    out_specs=[_spec_row, _spec_row],
    out_shape=[_row_out, _row_out],
)

_cb = pl.pallas_call(
    _cb_body, grid=(TC_GRID,),
    in_specs=[_spec_p, _spec_dp, _spec_row, _spec_b, _spec_b, _spec_w,
              _spec_b, _spec_w, _spec_b],
    out_specs=[_spec_row, _spec_row],
    out_shape=[_row_out, _row_out],
)

_cd = pl.pallas_call(
    _cd_body, grid=(TC_GRID,),
    in_specs=[_spec_p, _spec_dp, _spec_row, _spec_b, _spec_b, _spec_w,
              _spec_b],
    out_specs=_spec_row,
    out_shape=_row_out,
)


def kernel(x, edge_index, params):
    f32 = jnp.float32
    x_pad = jnp.zeros((N_PAD, D), f32).at[:N_NODES].set(x)
    ei = edge_index.astype(jnp.int32)
    packed = ei[0] | (ei[1] << 16)
    padv = N_NODES + jnp.arange(240, dtype=jnp.int32)
    padv = padv | (padv << 16)

    def tile_pad(arr2d, pad_per):
        pad = jnp.broadcast_to(jnp.tile(padv, pad_per // 240)[None, :],
                               (arr2d.shape[0], pad_per))
        return jnp.concatenate([arr2d, pad], axis=1)

    comb = tile_pad(packed.reshape(NC * NS, N_EDGES // (NC * NS)),
                    E_PAD // (NC * NS) - N_EDGES // (NC * NS)
                    ).reshape(NC, NS, NCHUNK, AGG_CHUNK)
    comb_deg = tile_pad(packed.reshape(NS, N_EDGES // NS),
                        E_PAD // NS - N_EDGES // NS
                        ).reshape(NS, NCHUNK_DEG, CHUNK)
    z128 = jnp.zeros((SLAB, D), f32)
    ones128 = jnp.ones((CHUNK, D), f32)

    def b(name):
        return params[name].reshape(1, D)

    dp = _deg_call(comb_deg, ones128, z128)
    nodes, h = _ab0(x_pad, dp, params["W_embed"], b("b_embed"),
                    params["W_mlp_0_0"], b("b_mlp_0_0"),
                    params["W_mlp_0_1"], b("b_mlp_0_1"))
    for s in range(3):
        p = _agg_call(comb, h, z128)
        lns = params[f"ln_scale_{s}"].reshape(1, D)
        lnb = params[f"ln_bias_{s}"].reshape(1, D)
        if s < 2:
            nodes, h = _cb(p, dp, nodes, lns, lnb,
                           params[f"W_mlp_{s + 1}_0"], b(f"b_mlp_{s + 1}_0"),
                           params[f"W_mlp_{s + 1}_1"], b(f"b_mlp_{s + 1}_1"))
        else:
            out = _cd(p, dp, nodes, lns, lnb, params["W_dec"], b("b_dec"))
    return out[:N_NODES]


# speedup vs baseline: 1.1255x; 1.1255x over previous
import functools

import jax
import jax.numpy as jnp
from jax import lax
from jax.experimental import pallas as pl
from jax.experimental.pallas import tpu as pltpu
from jax.experimental.pallas import tpu_sc as plsc

N_NODES = 10000
D = 128
N_PAD = 10240
N_EDGES = 320000
NC = 2
NS = 16
CHUNK = 128
AGG_CHUNK = 128
E_PAD = 327680
NCHUNK = E_PAD // (NC * NS * AGG_CHUNK)
NCHUNK_DEG = E_PAD // (NS * CHUNK)
SLAB = N_PAD // NS
TC_BLK = 640
TC_GRID = N_PAD // TC_BLK


def _deg_body(comb_hbm, ones_hbm, z_hbm, out_hbm,
              comb_v, idx_a, idx_b, ones_v, acc, sem_a, sem_b):
    cid = lax.axis_index("c")
    sid = lax.axis_index("s")
    pltpu.sync_copy(comb_hbm.at[sid], comb_v)
    pltpu.sync_copy(ones_hbm, ones_v)
    row0 = sid * SLAB
    pltpu.sync_copy(z_hbm, acc.at[pl.ds(row0, SLAB)])
    plsc.subcore_barrier()

    def extract(j, dst):
        for k in range(CHUNK // 16):
            v = comb_v[j, pl.ds(k * 16, 16)]
            lo = lax.bitwise_and(v, 0xFFFF)
            hi = lax.shift_right_logical(v, 16)
            dst[0, pl.ds(k * 16, 16)] = jnp.where(cid == 0, lo, hi)

    def scat(idx, sem):
        pltpu.async_copy(ones_v, acc.at[idx.at[0]], sem, add=True)

    def swait(idx, sem):
        pltpu.make_async_copy(ones_v, acc.at[idx.at[0]], sem).wait()

    extract(0, idx_a)
    scat(idx_a, sem_a)

    def body(jj, carry):
        j0 = 2 * jj
        extract(j0 + 1, idx_b)
        scat(idx_b, sem_b)
        swait(idx_a, sem_a)

        @pl.when(j0 + 2 < NCHUNK_DEG)
        def _():
            extract(j0 + 2, idx_a)
            scat(idx_a, sem_a)

        swait(idx_b, sem_b)
        return carry

    lax.fori_loop(0, NCHUNK_DEG // 2, body, 0)
    plsc.subcore_barrier()
    pltpu.sync_copy(acc.at[pl.ds(row0, SLAB)],
                    out_hbm.at[cid, pl.ds(row0, SLAB)])


def _agg_body(comb_hbm, h_hbm, z_hbm, out_hbm,
              comb_v, idx_a, idx_b, rows0, rows1, acc_sh, sem0, sem1):
    cid = lax.axis_index("c")
    sid = lax.axis_index("s")
    pltpu.sync_copy(comb_hbm.at[cid, sid], comb_v)
    row0 = sid * SLAB
    pltpu.sync_copy(z_hbm, acc_sh.at[pl.ds(row0, SLAB)])
    plsc.subcore_barrier()

    def extract(j, dst):
        for k in range(AGG_CHUNK // 16):
            v = comb_v[j, pl.ds(k * 16, 16)]
            dst[0, pl.ds(k * 16, 16)] = lax.bitwise_and(v, 0xFFFF)
            dst[1, pl.ds(k * 16, 16)] = lax.shift_right_logical(v, 16)

    def gather(idx, buf, sem):
        pltpu.async_copy(h_hbm.at[idx.at[0]], buf, sem)

    def gwait(idx, buf, sem):
        pltpu.make_async_copy(h_hbm.at[idx.at[0]], buf, sem).wait()

    extract(0, idx_a)
    gather(idx_a, rows0, sem0)

    def body(jj, carry):
        j0 = 2 * jj
        extract(j0 + 1, idx_b)
        gather(idx_b, rows1, sem1)
        gwait(idx_a, rows0, sem0)
        pltpu.sync_copy(rows0, acc_sh.at[idx_a.at[1]], add=True)

        @pl.when(j0 + 2 < NCHUNK)
        def _():
            extract(j0 + 2, idx_a)
            gather(idx_a, rows0, sem0)

        gwait(idx_b, rows1, sem1)
        pltpu.sync_copy(rows1, acc_sh.at[idx_b.at[1]], add=True)
        return carry

    lax.fori_loop(0, NCHUNK // 2, body, 0)
    plsc.subcore_barrier()
    pltpu.sync_copy(acc_sh.at[pl.ds(row0, SLAB)],
                    out_hbm.at[cid, pl.ds(row0, SLAB)])


def _sc_mesh():
    return plsc.VectorSubcoreMesh(core_axis_name="c", subcore_axis_name="s")


def _deg_call(comb_deg, ones128, z128):
    return pl.kernel(
        _deg_body,
        out_type=jax.ShapeDtypeStruct((NC, N_PAD, D), jnp.float32),
        mesh=_sc_mesh(),
        scratch_types=[
            pltpu.VMEM((NCHUNK_DEG, CHUNK), jnp.int32),
            pltpu.VMEM((1, CHUNK), jnp.int32),
            pltpu.VMEM((1, CHUNK), jnp.int32),
            pltpu.VMEM((CHUNK, D), jnp.float32),
            pltpu.VMEM_SHARED((N_PAD, D), jnp.float32),
            pltpu.SemaphoreType.DMA,
            pltpu.SemaphoreType.DMA,
        ],
    )(comb_deg, ones128, z128)


def _agg_call(comb, h, z128):
    return pl.kernel(
        _agg_body,
        out_type=jax.ShapeDtypeStruct((NC, N_PAD, D), jnp.float32),
        mesh=_sc_mesh(),
        scratch_types=[
            pltpu.VMEM((NCHUNK, AGG_CHUNK), jnp.int32),
            pltpu.VMEM((2, AGG_CHUNK), jnp.int32),
            pltpu.VMEM((2, AGG_CHUNK), jnp.int32),
            pltpu.VMEM((AGG_CHUNK, D), jnp.float32),
            pltpu.VMEM((AGG_CHUNK, D), jnp.float32),
            pltpu.VMEM_SHARED((N_PAD, D), jnp.float32),
            pltpu.SemaphoreType.DMA,
            pltpu.SemaphoreType.DMA,
        ],
    )(comb, h, z128)


def _inv_sqrt_deg(dp, kind):
    deg = dp[kind, :, 0:1].astype(jnp.float32)
    return lax.rsqrt(jnp.maximum(deg, 1.0))


def _mlp(h, w0, b0, w1, b1):
    h = jnp.maximum(jnp.dot(h, w0, preferred_element_type=jnp.float32) + b0, 0.0)
    h = jnp.maximum(jnp.dot(h, w1, preferred_element_type=jnp.float32) + b1, 0.0)
    return h


def _post(p, dp, nodes, lns, lnb):
    agg = (p[0] + p[1]) * _inv_sqrt_deg(dp, 1)
    nd = agg + nodes
    mean = jnp.mean(nd, axis=-1, keepdims=True)
    c = nd - mean
    var = jnp.mean(c * c, axis=-1, keepdims=True)
    return c * lax.rsqrt(var + 1e-6) * lns + lnb


def _ab0_body(x_ref, dp_ref, we, be, w0, b0, w1, b1, nodes_ref, h_ref):
    nodes = jnp.dot(x_ref[...], we[...],
                    preferred_element_type=jnp.float32) + be[...]
    h = _mlp(nodes, w0[...], b0[...], w1[...], b1[...])
    nodes_ref[...] = nodes
    h_ref[...] = h * _inv_sqrt_deg(dp_ref[...], 0)


def _cb_body(p_ref, dp_ref, nodes_ref, lns, lnb, w0, b0, w1, b1,
             nodes_out, h_out):
    dp = dp_ref[...]
    nodes_new = _post(p_ref[...], dp, nodes_ref[...], lns[...], lnb[...])
    h = _mlp(nodes_new, w0[...], b0[...], w1[...], b1[...])
    nodes_out[...] = nodes_new
    h_out[...] = h * _inv_sqrt_deg(dp, 0)


def _cd_body(p_ref, dp_ref, nodes_ref, lns, lnb, wd, bd, out_ref):
    nodes_new = _post(p_ref[...], dp_ref[...], nodes_ref[...], lns[...], lnb[...])
    out_ref[...] = jnp.dot(nodes_new, wd[...],
                           preferred_element_type=jnp.float32) + bd[...]


_spec_row = pl.BlockSpec((TC_BLK, D), lambda i: (i, 0))
_spec_w = pl.BlockSpec((D, D), lambda i: (0, 0))
_spec_b = pl.BlockSpec((1, D), lambda i: (0, 0))
_spec_dp = pl.BlockSpec((NC, TC_BLK, D), lambda i: (0, i, 0))
_spec_p = pl.BlockSpec((NC, TC_BLK, D), lambda i: (0, i, 0))

_row_out = jax.ShapeDtypeStruct((N_PAD, D), jnp.float32)

_ab0 = pl.pallas_call(
    _ab0_body, grid=(TC_GRID,),
    in_specs=[_spec_row, _spec_dp, _spec_w, _spec_b, _spec_w, _spec_b,
              _spec_w, _spec_b],
    out_specs=[_spec_row, _spec_row],
    out_shape=[_row_out, _row_out],
)

_cb = pl.pallas_call(
    _cb_body, grid=(TC_GRID,),
    in_specs=[_spec_p, _spec_dp, _spec_row, _spec_b, _spec_b, _spec_w,
              _spec_b, _spec_w, _spec_b],
    out_specs=[_spec_row, _spec_row],
    out_shape=[_row_out, _row_out],
)

_cd = pl.pallas_call(
    _cd_body, grid=(TC_GRID,),
    in_specs=[_spec_p, _spec_dp, _spec_row, _spec_b, _spec_b, _spec_w,
              _spec_b],
    out_specs=_spec_row,
    out_shape=_row_out,
)


def kernel(x, edge_index, params):
    f32 = jnp.float32
    x_pad = jnp.zeros((N_PAD, D), f32).at[:N_NODES].set(x)
    ei = edge_index.astype(jnp.int32)
    packed = ei[0] | (ei[1] << 16)
    padv = N_NODES + jnp.arange(240, dtype=jnp.int32)
    padv = padv | (padv << 16)

    def tile_pad(arr2d, pad_per):
        pad = jnp.broadcast_to(jnp.tile(padv, pad_per // 240)[None, :],
                               (arr2d.shape[0], pad_per))
        return jnp.concatenate([arr2d, pad], axis=1)

    comb = tile_pad(packed.reshape(NC * NS, N_EDGES // (NC * NS)),
                    E_PAD // (NC * NS) - N_EDGES // (NC * NS)
                    ).reshape(NC, NS, NCHUNK, AGG_CHUNK)
    comb_deg = tile_pad(packed.reshape(NS, N_EDGES // NS),
                        E_PAD // NS - N_EDGES // NS
                        ).reshape(NS, NCHUNK_DEG, CHUNK)
    z128 = jnp.zeros((SLAB, D), f32)
    ones128 = jnp.ones((CHUNK, D), f32)

    def b(name):
        return params[name].reshape(1, D)

    dp = _deg_call(comb_deg, ones128, z128)
    nodes, h = _ab0(x_pad, dp, params["W_embed"], b("b_embed"),
                    params["W_mlp_0_0"], b("b_mlp_0_0"),
                    params["W_mlp_0_1"], b("b_mlp_0_1"))
    for s in range(3):
        p = _agg_call(comb, h, z128)
        lns = params[f"ln_scale_{s}"].reshape(1, D)
        lnb = params[f"ln_bias_{s}"].reshape(1, D)
        if s < 2:
            nodes, h = _cb(p, dp, nodes, lns, lnb,
                           params[f"W_mlp_{s + 1}_0"], b(f"b_mlp_{s + 1}_0"),
                           params[f"W_mlp_{s + 1}_1"], b(f"b_mlp_{s + 1}_1"))
        else:
            out = _cd(p, dp, nodes, lns, lnb, params["W_dec"], b("b_dec"))
    return out[:N_NODES]
